# Initial kernel scaffold; baseline (speedup 1.0000x reference)
import jax
import jax.numpy as jnp
from jax.experimental import pallas as pl


def kernel(edge_feat, switch, edge_src, edge_dst, species):
    raise NotImplementedError("write your pallas kernel here")


# speedup vs baseline: 4.8216x; 4.8216x over previous
import functools

import jax
import jax.numpy as jnp
from jax import lax
from jax.experimental import pallas as pl
from jax.experimental.pallas import tpu as pltpu
from jax.experimental.pallas import tpu_sc as plsc

N_NODES = 10000
N_EDGES = 320000
D = 128

NUM_CORES = 2
NUM_SUBCORES = 16
NUM_WORKERS = NUM_CORES * NUM_SUBCORES
EDGES_PER_WORKER = N_EDGES // NUM_WORKERS
GROUP = 80
GROUPS_PER_WORKER = EDGES_PER_WORKER // GROUP
NCHUNK = 80
NUM_CHUNKS = N_NODES // NCHUNK


def _sc_scatter(feat_hbm, sw_hbm, src_hbm, dst_hbm):
    mesh = plsc.VectorSubcoreMesh(core_axis_name="c", subcore_axis_name="s")

    @functools.partial(
        pl.kernel,
        out_type=jax.ShapeDtypeStruct((NUM_CORES, N_NODES, D), jnp.float32),
        mesh=mesh,
        scratch_types=[
            pltpu.MemorySpace.VMEM_SHARED((N_NODES, D), jnp.float32),
            pltpu.VMEM((GROUP, D), jnp.float32),
            pltpu.VMEM((GROUP,), jnp.float32),
            pltpu.VMEM((GROUPS_PER_WORKER, GROUP), jnp.int32),
            pltpu.VMEM((GROUPS_PER_WORKER, GROUP), jnp.int32),
        ],
    )
    def run(feat, sw, src3d, dst3d, out, acc, fbuf, swg, sbuf, dbuf):
        c = lax.axis_index("c")
        s = lax.axis_index("s")
        w = c * NUM_SUBCORES + s
        ebase = w * EDGES_PER_WORKER

        def zrow(r, _):
            zero = jnp.zeros((16,), jnp.float32)
            for j in range(D // 16):
                fbuf[r, pl.ds(j * 16, 16)] = zero
            return 0

        lax.fori_loop(0, NCHUNK, zrow, 0)
        for k in range((NUM_CHUNKS + NUM_SUBCORES - 1) // NUM_SUBCORES):
            ch = k * NUM_SUBCORES + s

            @pl.when(ch < NUM_CHUNKS)
            def _():
                pltpu.sync_copy(fbuf, acc.at[pl.ds(ch * NCHUNK, NCHUNK)])

        pltpu.sync_copy(src3d.at[w], sbuf)
        pltpu.sync_copy(dst3d.at[w], dbuf)

        plsc.subcore_barrier()

        def group_body(g, _):
            pltpu.sync_copy(feat.at[pl.ds(ebase + g * GROUP, GROUP)], fbuf)
            pltpu.sync_copy(sw.at[pl.ds(ebase + g * GROUP, GROUP)], swg)

            def blk_body(b, _):
                sws = swg[pl.ds(b * 16, 16)]
                for i in range(16):
                    row = b * 16 + i
                    swb = jnp.full((16,), sws[i], jnp.float32)
                    for j in range(D // 16):
                        fbuf[row, pl.ds(j * 16, 16)] = (
                            fbuf[row, pl.ds(j * 16, 16)] * swb
                        )
                return 0

            lax.fori_loop(0, GROUP // 16, blk_body, 0)

            pltpu.sync_copy(fbuf, acc.at[sbuf.at[g]], add=True)
            pltpu.sync_copy(fbuf, acc.at[dbuf.at[g]], add=True)
            return 0

        lax.fori_loop(0, GROUPS_PER_WORKER, group_body, 0)

        plsc.subcore_barrier()

        for k in range((NUM_CHUNKS + NUM_SUBCORES - 1) // NUM_SUBCORES):
            ch = k * NUM_SUBCORES + s

            @pl.when(ch < NUM_CHUNKS)
            def _():
                pltpu.sync_copy(
                    acc.at[pl.ds(ch * NCHUNK, NCHUNK)],
                    out.at[c, pl.ds(ch * NCHUNK, NCHUNK)],
                )

    return run(feat_hbm, sw_hbm, src_hbm, dst_hbm)


def _combine_kernel(a_ref, b_ref, o_ref):
    o_ref[...] = a_ref[...] + b_ref[...]


def _combine(p0, p1):
    blk = 2000
    return pl.pallas_call(
        _combine_kernel,
        grid=(N_NODES // blk,),
        in_specs=[
            pl.BlockSpec((blk, D), lambda i: (i, 0)),
            pl.BlockSpec((blk, D), lambda i: (i, 0)),
        ],
        out_specs=pl.BlockSpec((blk, D), lambda i: (i, 0)),
        out_shape=jax.ShapeDtypeStruct((N_NODES, D), jnp.float32),
    )(p0, p1)


@jax.jit
def kernel(edge_feat, switch, edge_src, edge_dst, species):
    del species
    src3d = jnp.asarray(edge_src, jnp.int32).reshape(
        NUM_WORKERS, GROUPS_PER_WORKER, GROUP
    )
    dst3d = jnp.asarray(edge_dst, jnp.int32).reshape(
        NUM_WORKERS, GROUPS_PER_WORKER, GROUP
    )
    partials = _sc_scatter(edge_feat, switch, src3d, dst3d)
    return _combine(partials[0], partials[1])


# speedup vs baseline: 7.9319x; 1.6451x over previous
import functools

import jax
import jax.numpy as jnp
from jax import lax
from jax.experimental import pallas as pl
from jax.experimental.pallas import tpu as pltpu
from jax.experimental.pallas import tpu_sc as plsc

N_NODES = 10000
N_EDGES = 320000
D = 128

NUM_CORES = 2
NUM_SUBCORES = 16
NUM_WORKERS = NUM_CORES * NUM_SUBCORES
EDGES_PER_WORKER = N_EDGES // NUM_WORKERS
GROUP = 80
NGROUPS = EDGES_PER_WORKER // GROUP
NBUF = 3
NCHUNK = 80
NUM_CHUNKS = N_NODES // NCHUNK


def _sc_scatter(feat_hbm, sw_hbm, src_hbm, dst_hbm, zeros_hbm):
    mesh = plsc.VectorSubcoreMesh(core_axis_name="c", subcore_axis_name="s")

    @functools.partial(
        pl.kernel,
        out_type=jax.ShapeDtypeStruct((NUM_CORES, N_NODES, D), jnp.float32),
        mesh=mesh,
        scratch_types=[
            pltpu.MemorySpace.VMEM_SHARED((N_NODES, D), jnp.float32),
            [pltpu.VMEM((GROUP, D), jnp.float32) for _ in range(NBUF)],
            [pltpu.VMEM((1, GROUP), jnp.float32) for _ in range(NBUF)],
            [pltpu.VMEM((1, GROUP), jnp.int32) for _ in range(NBUF)],
            [pltpu.VMEM((1, GROUP), jnp.int32) for _ in range(NBUF)],
            [pltpu.SemaphoreType.DMA for _ in range(NBUF)],
            [pltpu.SemaphoreType.DMA for _ in range(NBUF)],
        ],
    )
    def run(feat, sw4d, src4d, dst4d, zeros, out, acc, fbufs, swbufs, sbufs,
            dbufs, lsems, ssems):
        c = lax.axis_index("c")
        s = lax.axis_index("s")
        w = c * NUM_SUBCORES + s
        ebase = w * EDGES_PER_WORKER

        def issue_loads(b, g):
            pltpu.async_copy(feat.at[pl.ds(ebase + g * GROUP, GROUP)],
                             fbufs[b], lsems[b])
            pltpu.async_copy(sw4d.at[w, g], swbufs[b], lsems[b])
            pltpu.async_copy(src4d.at[w, g], sbufs[b], lsems[b])
            pltpu.async_copy(dst4d.at[w, g], dbufs[b], lsems[b])

        def wait_loads(b, g):
            pltpu.make_async_copy(feat.at[pl.ds(ebase + g * GROUP, GROUP)],
                                  fbufs[b], lsems[b]).wait()
            pltpu.make_async_copy(sw4d.at[w, g], swbufs[b], lsems[b]).wait()
            pltpu.make_async_copy(src4d.at[w, g], sbufs[b], lsems[b]).wait()
            pltpu.make_async_copy(dst4d.at[w, g], dbufs[b], lsems[b]).wait()

        def issue_scatters(b):
            pltpu.async_copy(fbufs[b], acc.at[sbufs[b].at[0]], ssems[b],
                             add=True)
            pltpu.async_copy(fbufs[b], acc.at[dbufs[b].at[0]], ssems[b],
                             add=True)

        def wait_scatters(b):
            pltpu.make_async_copy(fbufs[b], acc.at[sbufs[b].at[0]],
                                  ssems[b]).wait()
            pltpu.make_async_copy(fbufs[b], acc.at[dbufs[b].at[0]],
                                  ssems[b]).wait()

        def scale(b):
            def blk_body(bb, _):
                sws = swbufs[b][0, pl.ds(bb * 16, 16)]
                for i in range(16):
                    row = bb * 16 + i
                    swb = jnp.full((16,), sws[i], jnp.float32)
                    for j in range(D // 16):
                        fbufs[b][row, pl.ds(j * 16, 16)] = (
                            fbufs[b][row, pl.ds(j * 16, 16)] * swb
                        )
                return 0

            lax.fori_loop(0, GROUP // 16, blk_body, 0)

        issue_loads(0, 0)
        issue_loads(1, 1)

        for k in range((NUM_CHUNKS + NUM_SUBCORES - 1) // NUM_SUBCORES):
            ch = k * NUM_SUBCORES + s

            @pl.when(ch < NUM_CHUNKS)
            def _():
                pltpu.sync_copy(zeros, acc.at[pl.ds(ch * NCHUNK, NCHUNK)])

        plsc.subcore_barrier()

        def body(k, _):
            for sct in range(NBUF):
                g = k * NBUF + sct
                b2 = (sct + 2) % NBUF

                @pl.when(g < NGROUPS)
                def _():
                    wait_loads(sct, g)
                    scale(sct)
                    issue_scatters(sct)

                    @pl.when(g >= 1)
                    def _():
                        wait_scatters(b2)

                    @pl.when(g + 2 < NGROUPS)
                    def _():
                        issue_loads(b2, g + 2)

            return 0

        lax.fori_loop(0, (NGROUPS + NBUF - 1) // NBUF, body, 0)

        wait_scatters((NGROUPS - 1) % NBUF)

        plsc.subcore_barrier()

        for k in range((NUM_CHUNKS + NUM_SUBCORES - 1) // NUM_SUBCORES):
            ch = k * NUM_SUBCORES + s

            @pl.when(ch < NUM_CHUNKS)
            def _():
                pltpu.sync_copy(
                    acc.at[pl.ds(ch * NCHUNK, NCHUNK)],
                    out.at[c, pl.ds(ch * NCHUNK, NCHUNK)],
                )

    return run(feat_hbm, sw_hbm, src_hbm, dst_hbm, zeros_hbm)


def _combine_kernel(a_ref, b_ref, o_ref):
    o_ref[...] = a_ref[...] + b_ref[...]


def _combine(p0, p1):
    blk = 2000
    return pl.pallas_call(
        _combine_kernel,
        grid=(N_NODES // blk,),
        in_specs=[
            pl.BlockSpec((blk, D), lambda i: (i, 0)),
            pl.BlockSpec((blk, D), lambda i: (i, 0)),
        ],
        out_specs=pl.BlockSpec((blk, D), lambda i: (i, 0)),
        out_shape=jax.ShapeDtypeStruct((N_NODES, D), jnp.float32),
    )(p0, p1)


@jax.jit
def kernel(edge_feat, switch, edge_src, edge_dst, species):
    del species
    shape4d = (NUM_WORKERS, NGROUPS, 1, GROUP)
    sw4d = switch.reshape(shape4d)
    src4d = jnp.asarray(edge_src, jnp.int32).reshape(shape4d)
    dst4d = jnp.asarray(edge_dst, jnp.int32).reshape(shape4d)
    zeros = jnp.zeros((NCHUNK, D), jnp.float32)
    partials = _sc_scatter(edge_feat, sw4d, src4d, dst4d, zeros)
    return _combine(partials[0], partials[1])


# speedup vs baseline: 8.7296x; 1.1006x over previous
import functools

import jax
import jax.numpy as jnp
from jax import lax
from jax.experimental import pallas as pl
from jax.experimental.pallas import tpu as pltpu
from jax.experimental.pallas import tpu_sc as plsc

N_NODES = 10000
N_EDGES = 320000
D = 128

NUM_CORES = 2
NUM_SUBCORES = 16
NUM_WORKERS = NUM_CORES * NUM_SUBCORES
EDGES_PER_WORKER = N_EDGES // NUM_WORKERS
GROUP = 80
NGROUPS = EDGES_PER_WORKER // GROUP
NBUF = 3
NCHUNK = 80
NUM_CHUNKS = N_NODES // NCHUNK


def _sc_scatter(feat_hbm, sw_hbm, src_hbm, dst_hbm, zeros_hbm):
    mesh = plsc.VectorSubcoreMesh(core_axis_name="c", subcore_axis_name="s")

    @functools.partial(
        pl.kernel,
        out_type=jax.ShapeDtypeStruct((NUM_CORES, N_NODES, D), jnp.float32),
        mesh=mesh,
        scratch_types=[
            pltpu.MemorySpace.VMEM_SHARED((N_NODES, D), jnp.float32),
            [pltpu.VMEM((GROUP, D), jnp.float32) for _ in range(NBUF)],
            [pltpu.VMEM((GROUP,), jnp.float32) for _ in range(NBUF)],
            [pltpu.VMEM((GROUP,), jnp.int32) for _ in range(NBUF)],
            [pltpu.VMEM((GROUP,), jnp.int32) for _ in range(NBUF)],
            [pltpu.SemaphoreType.DMA for _ in range(NBUF)],
            [pltpu.SemaphoreType.DMA for _ in range(NBUF)],
        ],
    )
    def run(feat, sw1d, src1d, dst1d, zeros, out, acc, fbufs, swbufs, sbufs,
            dbufs, lsems, ssems):
        c = lax.axis_index("c")
        s = lax.axis_index("s")
        w = c * NUM_SUBCORES + s
        ebase = w * EDGES_PER_WORKER

        def issue_loads(b, g):
            e0 = ebase + g * GROUP
            pltpu.async_copy(feat.at[pl.ds(e0, GROUP)], fbufs[b], lsems[b])
            pltpu.async_copy(sw1d.at[pl.ds(e0, GROUP)], swbufs[b], lsems[b])
            pltpu.async_copy(src1d.at[pl.ds(e0, GROUP)], sbufs[b], lsems[b])
            pltpu.async_copy(dst1d.at[pl.ds(e0, GROUP)], dbufs[b], lsems[b])

        def wait_loads(b, g):
            e0 = ebase + g * GROUP
            pltpu.make_async_copy(feat.at[pl.ds(e0, GROUP)], fbufs[b],
                                  lsems[b]).wait()
            pltpu.make_async_copy(sw1d.at[pl.ds(e0, GROUP)], swbufs[b],
                                  lsems[b]).wait()
            pltpu.make_async_copy(src1d.at[pl.ds(e0, GROUP)], sbufs[b],
                                  lsems[b]).wait()
            pltpu.make_async_copy(dst1d.at[pl.ds(e0, GROUP)], dbufs[b],
                                  lsems[b]).wait()

        def issue_scatters(b):
            pltpu.async_copy(fbufs[b], acc.at[sbufs[b]], ssems[b], add=True)
            pltpu.async_copy(fbufs[b], acc.at[dbufs[b]], ssems[b], add=True)

        def wait_scatters(b):
            pltpu.make_async_copy(fbufs[b], acc.at[sbufs[b]], ssems[b]).wait()
            pltpu.make_async_copy(fbufs[b], acc.at[dbufs[b]], ssems[b]).wait()

        def scale(b):
            def blk_body(bb, _):
                sws = swbufs[b][pl.ds(bb * 16, 16)]
                for i in range(16):
                    row = bb * 16 + i
                    swb = jnp.full((16,), sws[i], jnp.float32)
                    for j in range(D // 16):
                        fbufs[b][row, pl.ds(j * 16, 16)] = (
                            fbufs[b][row, pl.ds(j * 16, 16)] * swb
                        )
                return 0

            lax.fori_loop(0, GROUP // 16, blk_body, 0)

        issue_loads(0, 0)
        issue_loads(1, 1)

        for k in range((NUM_CHUNKS + NUM_SUBCORES - 1) // NUM_SUBCORES):
            ch = k * NUM_SUBCORES + s

            @pl.when(ch < NUM_CHUNKS)
            def _():
                pltpu.sync_copy(zeros, acc.at[pl.ds(ch * NCHUNK, NCHUNK)])

        plsc.subcore_barrier()

        def body(k, _):
            for sct in range(NBUF):
                g = k * NBUF + sct
                b2 = (sct + 2) % NBUF

                @pl.when(g < NGROUPS)
                def _():
                    wait_loads(sct, g)
                    scale(sct)
                    issue_scatters(sct)

                    @pl.when(g >= 1)
                    def _():
                        wait_scatters(b2)

                    @pl.when(g + 2 < NGROUPS)
                    def _():
                        issue_loads(b2, g + 2)

            return 0

        lax.fori_loop(0, (NGROUPS + NBUF - 1) // NBUF, body, 0)

        wait_scatters((NGROUPS - 1) % NBUF)

        plsc.subcore_barrier()

        for k in range((NUM_CHUNKS + NUM_SUBCORES - 1) // NUM_SUBCORES):
            ch = k * NUM_SUBCORES + s

            @pl.when(ch < NUM_CHUNKS)
            def _():
                pltpu.sync_copy(
                    acc.at[pl.ds(ch * NCHUNK, NCHUNK)],
                    out.at[c, pl.ds(ch * NCHUNK, NCHUNK)],
                )

    return run(feat_hbm, sw_hbm, src_hbm, dst_hbm, zeros_hbm)


def _combine_kernel(a_ref, b_ref, o_ref):
    o_ref[...] = a_ref[...] + b_ref[...]


def _combine(p0, p1):
    blk = 2000
    return pl.pallas_call(
        _combine_kernel,
        grid=(N_NODES // blk,),
        in_specs=[
            pl.BlockSpec((blk, D), lambda i: (i, 0)),
            pl.BlockSpec((blk, D), lambda i: (i, 0)),
        ],
        out_specs=pl.BlockSpec((blk, D), lambda i: (i, 0)),
        out_shape=jax.ShapeDtypeStruct((N_NODES, D), jnp.float32),
    )(p0, p1)


@jax.jit
def kernel(edge_feat, switch, edge_src, edge_dst, species):
    del species
    src1d = jnp.asarray(edge_src, jnp.int32)
    dst1d = jnp.asarray(edge_dst, jnp.int32)
    zeros = jnp.zeros((NCHUNK, D), jnp.float32)
    partials = _sc_scatter(edge_feat, switch, src1d, dst1d, zeros)
    return _combine(partials[0], partials[1])


# speedup vs baseline: 8.9859x; 1.0294x over previous
import functools

import jax
import jax.numpy as jnp
from jax import lax
from jax.experimental import pallas as pl
from jax.experimental.pallas import tpu as pltpu
from jax.experimental.pallas import tpu_sc as plsc

N_NODES = 10000
N_EDGES = 320000
D = 128

NUM_CORES = 2
NUM_SUBCORES = 16
NUM_WORKERS = NUM_CORES * NUM_SUBCORES
EDGES_PER_WORKER = N_EDGES // NUM_WORKERS
GROUP = 80
NGROUPS = EDGES_PER_WORKER // GROUP
NBUF = 3
NCHUNK = 80
NUM_CHUNKS = N_NODES // NCHUNK


def _sc_scatter(feat_hbm, sw_hbm, src_hbm, dst_hbm, zeros_hbm):
    mesh = plsc.VectorSubcoreMesh(core_axis_name="c", subcore_axis_name="s")

    @functools.partial(
        pl.kernel,
        out_type=jax.ShapeDtypeStruct((NUM_CORES, N_NODES, D), jnp.float32),
        mesh=mesh,
        scratch_types=[
            pltpu.MemorySpace.VMEM_SHARED((N_NODES, D), jnp.float32),
            [pltpu.VMEM((GROUP, D), jnp.float32) for _ in range(NBUF)],
            [pltpu.VMEM((GROUP,), jnp.float32) for _ in range(NBUF)],
            [pltpu.VMEM((GROUP,), jnp.int32) for _ in range(NBUF)],
            [pltpu.VMEM((GROUP,), jnp.int32) for _ in range(NBUF)],
            [pltpu.SemaphoreType.DMA for _ in range(NBUF)],
            [pltpu.SemaphoreType.DMA for _ in range(NBUF)],
        ],
    )
    def run(feat, sw1d, src1d, dst1d, zeros, out, acc, fbufs, swbufs, sbufs,
            dbufs, lsems, ssems):
        c = lax.axis_index("c")
        s = lax.axis_index("s")
        w = c * NUM_SUBCORES + s
        ebase = w * EDGES_PER_WORKER

        def issue_loads(b, g):
            e0 = ebase + g * GROUP
            pltpu.async_copy(feat.at[pl.ds(e0, GROUP)], fbufs[b], lsems[b])
            pltpu.async_copy(sw1d.at[pl.ds(e0, GROUP)], swbufs[b], lsems[b])
            pltpu.async_copy(src1d.at[pl.ds(e0, GROUP)], sbufs[b], lsems[b])
            pltpu.async_copy(dst1d.at[pl.ds(e0, GROUP)], dbufs[b], lsems[b])

        def wait_loads(b, g):
            e0 = ebase + g * GROUP
            pltpu.make_async_copy(feat.at[pl.ds(e0, GROUP)], fbufs[b],
                                  lsems[b]).wait()
            pltpu.make_async_copy(sw1d.at[pl.ds(e0, GROUP)], swbufs[b],
                                  lsems[b]).wait()
            pltpu.make_async_copy(src1d.at[pl.ds(e0, GROUP)], sbufs[b],
                                  lsems[b]).wait()
            pltpu.make_async_copy(dst1d.at[pl.ds(e0, GROUP)], dbufs[b],
                                  lsems[b]).wait()

        def issue_scatters(b):
            pltpu.async_copy(fbufs[b], acc.at[sbufs[b]], ssems[b], add=True)
            pltpu.async_copy(fbufs[b], acc.at[dbufs[b]], ssems[b], add=True)

        def wait_scatters(b):
            pltpu.make_async_copy(fbufs[b], acc.at[sbufs[b]], ssems[b]).wait()
            pltpu.make_async_copy(fbufs[b], acc.at[dbufs[b]], ssems[b]).wait()

        def scale(b):
            def blk_body(bb, _):
                sws = swbufs[b][pl.ds(bb * 16, 16)]
                for i in range(16):
                    row = bb * 16 + i
                    swb = jnp.full((16,), sws[i], jnp.float32)
                    for j in range(D // 16):
                        fbufs[b][row, pl.ds(j * 16, 16)] = (
                            fbufs[b][row, pl.ds(j * 16, 16)] * swb
                        )
                return 0

            lax.fori_loop(0, GROUP // 16, blk_body, 0)

        issue_loads(0, 0)
        issue_loads(1, 1)

        nz = (NUM_CHUNKS + NUM_SUBCORES - 1) // NUM_SUBCORES
        for k in range(nz):
            ch = k * NUM_SUBCORES + s

            @pl.when(ch < NUM_CHUNKS)
            def _():
                pltpu.async_copy(zeros, acc.at[pl.ds(ch * NCHUNK, NCHUNK)],
                                 ssems[0])

        for k in range(nz):
            ch = k * NUM_SUBCORES + s

            @pl.when(ch < NUM_CHUNKS)
            def _():
                pltpu.make_async_copy(
                    zeros, acc.at[pl.ds(ch * NCHUNK, NCHUNK)], ssems[0]
                ).wait()

        plsc.subcore_barrier()

        def body(k, _):
            for sct in range(NBUF):
                g = k * NBUF + sct
                b2 = (sct + 2) % NBUF

                @pl.when(g < NGROUPS)
                def _():
                    wait_loads(sct, g)
                    scale(sct)
                    issue_scatters(sct)

                    @pl.when(g >= 1)
                    def _():
                        wait_scatters(b2)

                    @pl.when(g + 2 < NGROUPS)
                    def _():
                        issue_loads(b2, g + 2)

            return 0

        lax.fori_loop(0, (NGROUPS + NBUF - 1) // NBUF, body, 0)

        wait_scatters((NGROUPS - 1) % NBUF)

        plsc.subcore_barrier()

        for k in range((NUM_CHUNKS + NUM_SUBCORES - 1) // NUM_SUBCORES):
            ch = k * NUM_SUBCORES + s

            @pl.when(ch < NUM_CHUNKS)
            def _():
                pltpu.sync_copy(
                    acc.at[pl.ds(ch * NCHUNK, NCHUNK)],
                    out.at[c, pl.ds(ch * NCHUNK, NCHUNK)],
                )

    return run(feat_hbm, sw_hbm, src_hbm, dst_hbm, zeros_hbm)


def _combine_kernel(p_ref, o_ref):
    o_ref[...] = p_ref[0] + p_ref[1]


def _combine(partials):
    blk = 2000
    return pl.pallas_call(
        _combine_kernel,
        grid=(N_NODES // blk,),
        in_specs=[pl.BlockSpec((NUM_CORES, blk, D), lambda i: (0, i, 0))],
        out_specs=pl.BlockSpec((blk, D), lambda i: (i, 0)),
        out_shape=jax.ShapeDtypeStruct((N_NODES, D), jnp.float32),
    )(partials)


@jax.jit
def kernel(edge_feat, switch, edge_src, edge_dst, species):
    del species
    src1d = jnp.asarray(edge_src, jnp.int32)
    dst1d = jnp.asarray(edge_dst, jnp.int32)
    zeros = jnp.zeros((NCHUNK, D), jnp.float32)
    partials = _sc_scatter(edge_feat, switch, src1d, dst1d, zeros)
    return _combine(partials)


# speedup vs baseline: 9.4189x; 1.0482x over previous
import functools

import jax
import jax.numpy as jnp
from jax import lax
from jax.experimental import pallas as pl
from jax.experimental.pallas import tpu as pltpu
from jax.experimental.pallas import tpu_sc as plsc

N_NODES = 10000
N_EDGES = 320000
D = 128

NUM_CORES = 2
NUM_SUBCORES = 16
NUM_WORKERS = NUM_CORES * NUM_SUBCORES
EDGES_PER_WORKER = N_EDGES // NUM_WORKERS
GROUP = 40
NGROUPS = EDGES_PER_WORKER // GROUP
NBUF = 4
NCHUNK = 80
NUM_CHUNKS = N_NODES // NCHUNK


def _sc_scatter(feat_hbm, sw_hbm, src_hbm, dst_hbm, zeros_hbm):
    mesh = plsc.VectorSubcoreMesh(core_axis_name="c", subcore_axis_name="s")

    @functools.partial(
        pl.kernel,
        out_type=jax.ShapeDtypeStruct((NUM_CORES, N_NODES, D), jnp.float32),
        mesh=mesh,
        scratch_types=[
            pltpu.MemorySpace.VMEM_SHARED((N_NODES, D), jnp.float32),
            [pltpu.VMEM((GROUP, D), jnp.float32) for _ in range(NBUF)],
            [pltpu.VMEM((GROUP,), jnp.float32) for _ in range(NBUF)],
            [pltpu.VMEM((GROUP,), jnp.int32) for _ in range(NBUF)],
            [pltpu.VMEM((GROUP,), jnp.int32) for _ in range(NBUF)],
            [pltpu.SemaphoreType.DMA for _ in range(NBUF)],
            [pltpu.SemaphoreType.DMA for _ in range(NBUF)],
        ],
    )
    def run(feat, sw1d, src1d, dst1d, zeros, out, acc, fbufs, swbufs, sbufs,
            dbufs, lsems, ssems):
        c = lax.axis_index("c")
        s = lax.axis_index("s")
        w = c * NUM_SUBCORES + s
        ebase = w * EDGES_PER_WORKER

        def issue_loads(b, g):
            e0 = ebase + g * GROUP
            pltpu.async_copy(feat.at[pl.ds(e0, GROUP)], fbufs[b], lsems[b])
            pltpu.async_copy(sw1d.at[pl.ds(e0, GROUP)], swbufs[b], lsems[b])
            pltpu.async_copy(src1d.at[pl.ds(e0, GROUP)], sbufs[b], lsems[b])
            pltpu.async_copy(dst1d.at[pl.ds(e0, GROUP)], dbufs[b], lsems[b])

        def wait_loads(b, g):
            e0 = ebase + g * GROUP
            pltpu.make_async_copy(feat.at[pl.ds(e0, GROUP)], fbufs[b],
                                  lsems[b]).wait()
            pltpu.make_async_copy(sw1d.at[pl.ds(e0, GROUP)], swbufs[b],
                                  lsems[b]).wait()
            pltpu.make_async_copy(src1d.at[pl.ds(e0, GROUP)], sbufs[b],
                                  lsems[b]).wait()
            pltpu.make_async_copy(dst1d.at[pl.ds(e0, GROUP)], dbufs[b],
                                  lsems[b]).wait()

        def issue_scatters(b):
            pltpu.async_copy(fbufs[b], acc.at[sbufs[b]], ssems[b], add=True)
            pltpu.async_copy(fbufs[b], acc.at[dbufs[b]], ssems[b], add=True)

        def wait_scatters(b):
            pltpu.make_async_copy(fbufs[b], acc.at[sbufs[b]], ssems[b]).wait()
            pltpu.make_async_copy(fbufs[b], acc.at[dbufs[b]], ssems[b]).wait()

        def scale(b):
            def blk_body(bb, _):
                sws = swbufs[b][pl.ds(bb * 16, 16)]
                for i in range(16):
                    row = bb * 16 + i
                    swb = jnp.full((16,), sws[i], jnp.float32)
                    for j in range(D // 16):
                        fbufs[b][row, pl.ds(j * 16, 16)] = (
                            fbufs[b][row, pl.ds(j * 16, 16)] * swb
                        )
                return 0

            lax.fori_loop(0, GROUP // 16, blk_body, 0)

        for b in range(NBUF - 1):
            issue_loads(b, b)

        nz = (NUM_CHUNKS + NUM_SUBCORES - 1) // NUM_SUBCORES
        for k in range(nz):
            ch = k * NUM_SUBCORES + s

            @pl.when(ch < NUM_CHUNKS)
            def _():
                pltpu.async_copy(zeros, acc.at[pl.ds(ch * NCHUNK, NCHUNK)],
                                 ssems[0])

        for k in range(nz):
            ch = k * NUM_SUBCORES + s

            @pl.when(ch < NUM_CHUNKS)
            def _():
                pltpu.make_async_copy(
                    zeros, acc.at[pl.ds(ch * NCHUNK, NCHUNK)], ssems[0]
                ).wait()

        plsc.subcore_barrier()

        def body(k, _):
            for sct in range(NBUF):
                g = k * NBUF + sct
                b2 = (sct + NBUF - 1) % NBUF

                @pl.when(g < NGROUPS)
                def _():
                    wait_loads(sct, g)
                    scale(sct)
                    issue_scatters(sct)

                    @pl.when(g >= 1)
                    def _():
                        wait_scatters(b2)

                    @pl.when(g + NBUF - 1 < NGROUPS)
                    def _():
                        issue_loads(b2, g + NBUF - 1)

            return 0

        lax.fori_loop(0, (NGROUPS + NBUF - 1) // NBUF, body, 0)

        wait_scatters((NGROUPS - 1) % NBUF)

        plsc.subcore_barrier()

        for k in range((NUM_CHUNKS + NUM_SUBCORES - 1) // NUM_SUBCORES):
            ch = k * NUM_SUBCORES + s

            @pl.when(ch < NUM_CHUNKS)
            def _():
                pltpu.sync_copy(
                    acc.at[pl.ds(ch * NCHUNK, NCHUNK)],
                    out.at[c, pl.ds(ch * NCHUNK, NCHUNK)],
                )

    return run(feat_hbm, sw_hbm, src_hbm, dst_hbm, zeros_hbm)


def _combine_kernel(p_ref, o_ref):
    o_ref[...] = p_ref[0] + p_ref[1]


def _combine(partials):
    blk = 2000
    return pl.pallas_call(
        _combine_kernel,
        grid=(N_NODES // blk,),
        in_specs=[pl.BlockSpec((NUM_CORES, blk, D), lambda i: (0, i, 0))],
        out_specs=pl.BlockSpec((blk, D), lambda i: (i, 0)),
        out_shape=jax.ShapeDtypeStruct((N_NODES, D), jnp.float32),
    )(partials)


@jax.jit
def kernel(edge_feat, switch, edge_src, edge_dst, species):
    del species
    src1d = jnp.asarray(edge_src, jnp.int32)
    dst1d = jnp.asarray(edge_dst, jnp.int32)
    zeros = jnp.zeros((NCHUNK, D), jnp.float32)
    partials = _sc_scatter(edge_feat, switch, src1d, dst1d, zeros)
    return _combine(partials)
